# Initial kernel scaffold; baseline (speedup 1.0000x reference)
import jax
import jax.numpy as jnp
from jax.experimental import pallas as pl


def kernel(x, edge_index, W1, b1, W2, b2, W3, b3, g1, be1, g2, be2):
    raise NotImplementedError("write your pallas kernel here")


# speedup vs baseline: 11.1762x; 11.1762x over previous
import functools

import jax
import jax.numpy as jnp
from jax import lax
from jax.experimental import pallas as pl
from jax.experimental.pallas import tpu as pltpu
from jax.experimental.pallas import tpu_sc as plsc

_NC = 2
_NS = 16
_DW = 16


def _striped_copy(n, s, copy_fn):
    rpt8 = (n // _NS) // 8 * 8
    rem = n - _NS * rpt8
    copy_fn(pl.ds(s * rpt8, rpt8))
    if rem:
        @pl.when(s == _NS - 1)
        def _():
            copy_fn(pl.ds(_NS * rpt8, rem))


def _sc_degree(dst, init):
    e = dst.shape[0]
    n = init.shape[1]
    nw = _NC * _NS
    epw = e // nw
    k = 80
    nchunk = epw // k
    mesh = plsc.VectorSubcoreMesh(core_axis_name="c", subcore_axis_name="s", num_cores=_NC, num_subcores=_NS)

    @functools.partial(
        pl.kernel,
        out_type=jax.ShapeDtypeStruct((_NC, n, _DW), jnp.float32),
        mesh=mesh,
        scratch_types=[
            pltpu.VMEM((k,), jnp.int32),
            pltpu.VMEM((k, _DW), jnp.float32),
            pltpu.VMEM_SHARED((n, _DW), jnp.float32),
        ],
    )
    def deg_kernel(dst_hbm, init_hbm, out_hbm, dstv, ones, acc_ref):
        c = lax.axis_index("c")
        s = lax.axis_index("s")

        @pl.loop(0, k)
        def _(i):
            ones[i, :] = jnp.full((_DW,), 1.0, jnp.float32)

        _striped_copy(n, s, lambda sl: pltpu.sync_copy(
            init_hbm.at[c].at[sl], acc_ref.at[sl]))
        plsc.subcore_barrier()
        base = (c * _NS + s) * epw

        @pl.loop(0, nchunk)
        def _(j):
            pltpu.sync_copy(dst_hbm.at[pl.ds(base + j * k, k)], dstv)
            pltpu.sync_copy(ones, acc_ref.at[dstv], add=True)

        plsc.subcore_barrier()
        _striped_copy(n, s, lambda sl: pltpu.sync_copy(
            acc_ref.at[sl], out_hbm.at[c].at[sl]))

    return deg_kernel(dst, init)


def _sc_aggregate(hs, src, dst):
    n, h = hs.shape
    e = src.shape[0]
    nw = _NC * _NS
    epw = e // nw
    k = 80
    nchunk = epw // k
    mesh = plsc.VectorSubcoreMesh(core_axis_name="c", subcore_axis_name="s", num_cores=_NC, num_subcores=_NS)

    @functools.partial(
        pl.kernel,
        out_type=jax.ShapeDtypeStruct((_NC, n, h), jnp.float32),
        mesh=mesh,
        scratch_types=[
            pltpu.VMEM((k,), jnp.int32),
            pltpu.VMEM((k,), jnp.int32),
            pltpu.VMEM((k, h), jnp.float32),
            pltpu.VMEM_SHARED((n, h), jnp.float32),
            pltpu.SemaphoreType.DMA,
        ],
    )
    def agg_kernel(hs_hbm, src_hbm, dst_hbm, out_hbm, srcv, dstv, rows, acc_ref, sem):
        c = lax.axis_index("c")
        s = lax.axis_index("s")

        _striped_copy(n, s, lambda sl: pltpu.sync_copy(
            hs_hbm.at[sl], acc_ref.at[sl]))
        plsc.subcore_barrier()
        base = (c * _NS + s) * epw

        @pl.loop(0, nchunk)
        def _(j):
            off = base + j * k
            pltpu.sync_copy(src_hbm.at[pl.ds(off, k)], srcv)
            pltpu.sync_copy(dst_hbm.at[pl.ds(off, k)], dstv)
            pltpu.async_copy(hs_hbm.at[srcv], rows, sem).wait()
            pltpu.sync_copy(rows, acc_ref.at[dstv], add=True)

        plsc.subcore_barrier()
        _striped_copy(n, s, lambda sl: pltpu.sync_copy(
            acc_ref.at[sl], out_hbm.at[c].at[sl]))

    return agg_kernel(hs, src, dst)


def _tc_pre(x, w1, degp):
    n, d_in = x.shape
    h = w1.shape[1]

    def body(x_ref, w_ref, deg_ref, hs_ref, dinv_ref):
        deg = deg_ref[0] + deg_ref[1]
        dinv = lax.rsqrt(deg)
        dcol = dinv[:, 0:1]
        hh = jnp.dot(x_ref[...], w_ref[...], preferred_element_type=jnp.float32)
        hs_ref[...] = hh * dcol
        dinv_ref[...] = dcol

    return pl.pallas_call(
        body,
        out_shape=[
            jax.ShapeDtypeStruct((n, h), jnp.float32),
            jax.ShapeDtypeStruct((n, 1), jnp.float32),
        ],
    )(x, w1, degp)


def _tc_mid(p, hs, dinv, b, g, be, w_next):
    n, h = hs.shape

    def body(p_ref, hs_ref, dinv_ref, b_ref, g_ref, be_ref, w_ref, o_ref):
        dcol = dinv_ref[...]
        z = dcol * (p_ref[0] + p_ref[1] - hs_ref[...]) + b_ref[...][None, :]
        mean = jnp.mean(z, axis=0, keepdims=True)
        zc = z - mean
        var = jnp.mean(zc * zc, axis=0, keepdims=True)
        zn = g_ref[...][None, :] * zc * lax.rsqrt(var + 1e-5) + be_ref[...][None, :]
        a = jnp.maximum(zn, 0.0)
        o_ref[...] = jnp.dot(a, w_ref[...], preferred_element_type=jnp.float32) * dcol

    return pl.pallas_call(
        body,
        out_shape=jax.ShapeDtypeStruct((n, h), jnp.float32),
    )(p, hs, dinv, b, g, be, w_next)


def _tc_fin(p, hs, dinv, b):
    n, h = hs.shape

    def body(p_ref, hs_ref, dinv_ref, b_ref, o_ref):
        dcol = dinv_ref[...]
        o_ref[...] = dcol * (p_ref[0] + p_ref[1] - hs_ref[...]) + b_ref[...][None, :]

    return pl.pallas_call(
        body,
        out_shape=jax.ShapeDtypeStruct((n, h), jnp.float32),
    )(p, hs, dinv, b)


def kernel(x, edge_index, W1, b1, W2, b2, W3, b3, g1, be1, g2, be2):
    n = x.shape[0]
    src = edge_index[0]
    dst = edge_index[1]
    init = jnp.concatenate(
        [jnp.ones((1, n, _DW), jnp.float32), jnp.zeros((1, n, _DW), jnp.float32)]
    )
    degp = _sc_degree(dst, init)
    hs1, dinv = _tc_pre(x, W1, degp)
    p1 = _sc_aggregate(hs1, src, dst)
    hs2 = _tc_mid(p1, hs1, dinv, b1, g1, be1, W2)
    p2 = _sc_aggregate(hs2, src, dst)
    hs3 = _tc_mid(p2, hs2, dinv, b2, g2, be2, W3)
    p3 = _sc_aggregate(hs3, src, dst)
    return _tc_fin(p3, hs3, dinv, b3)
